# baseline (speedup 1.0000x reference)
import functools

import jax
import jax.numpy as jnp
from jax import lax
from jax.experimental import pallas as pl
from jax.experimental.pallas import tpu as pltpu
from jax.experimental.pallas import tpu_sc as plsc

N = 10000
D = 128
K = 4
E = 320000

NC = 2
NS = 16
NW = NC * NS
L = 16

NPAD = 10240
EP = 327680
EPW = EP // NW
C = 128
NCHUNK = EPW // C
NPS = NPAD // NS
NPW = NPAD // NW

_MESH = plsc.VectorSubcoreMesh(core_axis_name="c", subcore_axis_name="s")

ROWS_TC = 2000


NPA = 10016
NPSA = NPA // NS
DUMP = NPA - 1


def _k1_body(x_ref, w_ref, zk_ref, aux_ref):
    z = jnp.dot(x_ref[...], w_ref[...], preferred_element_type=jnp.float32)
    zk_ref[...] = z[:, : K * D].astype(jnp.bfloat16)
    aux_ref[...] = z[:, K * D :]


def _k1(x, wcat):
    return pl.pallas_call(
        _k1_body,
        grid=(N // ROWS_TC,),
        in_specs=[
            pl.BlockSpec((ROWS_TC, D), lambda i: (i, 0)),
            pl.BlockSpec((D, 6 * D), lambda i: (0, 0)),
        ],
        out_specs=[
            pl.BlockSpec((ROWS_TC, K * D), lambda i: (i, 0)),
            pl.BlockSpec((ROWS_TC, 2 * D), lambda i: (i, 0)),
        ],
        out_shape=[
            jax.ShapeDtypeStruct((N, K * D), jnp.bfloat16),
            jax.ShapeDtypeStruct((N, 2 * D), jnp.float32),
        ],
    )(x, wcat)


@functools.partial(
    pl.kernel,
    out_type=(
        jax.ShapeDtypeStruct((EP,), jnp.float32),
        jax.ShapeDtypeStruct((NW, NPAD), jnp.float32),
    ),
    mesh=_MESH,
    compiler_params=pltpu.CompilerParams(needs_layout_passes=False, use_tc_tiling_on_sc=False),
    scratch_types=[
        pltpu.VMEM((NPAD,), jnp.float32),
        pltpu.VMEM((NPAD,), jnp.float32),
        pltpu.VMEM((C,), jnp.int32),
        pltpu.VMEM((C,), jnp.int32),
        pltpu.VMEM((K, C), jnp.float32),
        pltpu.VMEM((C,), jnp.float32),
    ],
)
def _k2(src_hbm, dst_hbm, eat_hbm, s_hbm, zed_hbm,
        ex_hbm, dpart_hbm,
        s_v, den_v, src_v, dst_v, ea_v, ex_v):
    cid = lax.axis_index("c")
    sid = lax.axis_index("s")
    wid = sid * NC + cid
    pltpu.sync_copy(s_hbm, s_v)
    pltpu.sync_copy(zed_hbm, den_v)

    def chunk(g, carry):
        base = wid * EPW + g * C
        pltpu.sync_copy(src_hbm.at[pl.ds(base, C)], src_v)
        pltpu.sync_copy(dst_hbm.at[pl.ds(base, C)], dst_v)
        for k in range(K):
            pltpu.sync_copy(eat_hbm.at[k, pl.ds(base, C)], ea_v.at[k])
        for i in range(C // L):
            sl = pl.ds(i * L, L)
            sg = plsc.load_gather(s_v, [src_v[sl]])
            ebar = (ea_v[0, sl] + ea_v[1, sl] + ea_v[2, sl] + ea_v[3, sl]) * 0.25
            ex = jnp.exp(ebar * sg)
            ex_v[sl] = ex
            plsc.addupdate_scatter(den_v, [dst_v[sl]], ex)
        pltpu.sync_copy(ex_v, ex_hbm.at[pl.ds(base, C)])
        return carry

    lax.fori_loop(0, NCHUNK, chunk, 0)
    pltpu.sync_copy(den_v, dpart_hbm.at[wid])


@functools.partial(
    pl.kernel,
    out_type=jax.ShapeDtypeStruct((NPAD,), jnp.float32),
    mesh=_MESH,
    compiler_params=pltpu.CompilerParams(needs_layout_passes=False, use_tc_tiling_on_sc=False),
    scratch_types=[
        pltpu.VMEM((NW, NPW), jnp.float32),
        pltpu.VMEM((NPW,), jnp.float32),
    ],
)
def _k3(dpart_hbm, recip_hbm, part_v, acc_v):
    wid = lax.axis_index("s") * NC + lax.axis_index("c")
    for j in range(NW):
        pltpu.sync_copy(dpart_hbm.at[j, pl.ds(wid * NPW, NPW)], part_v.at[j])
    for i in range(NPW // L):
        sl = pl.ds(i * L, L)
        a = part_v[0, sl]
        for j in range(1, NW):
            a = a + part_v[j, sl]
        acc_v[sl] = 1.0 / (a + 1e-16)
    pltpu.sync_copy(acc_v, recip_hbm.at[pl.ds(wid * NPW, NPW)])


CB = 512


@functools.partial(
    pl.kernel,
    out_type=jax.ShapeDtypeStruct((EP,), jnp.float32),
    mesh=_MESH,
    compiler_params=pltpu.CompilerParams(needs_layout_passes=False, use_tc_tiling_on_sc=False),
    scratch_types=[
        pltpu.VMEM((NPAD,), jnp.float32),
        pltpu.VMEM((CB,), jnp.int32),
        pltpu.VMEM((CB,), jnp.float32),
        pltpu.VMEM((CB,), jnp.float32),
    ],
)
def _k3b(dst_hbm, ex_hbm, recip_hbm, att_hbm, recip_v, dst_v, ex_v, att_v):
    wid = lax.axis_index("s") * NC + lax.axis_index("c")
    pltpu.sync_copy(recip_hbm, recip_v)

    def chunk(g, carry):
        base = wid * EPW + g * CB
        pltpu.sync_copy(dst_hbm.at[pl.ds(base, CB)], dst_v)
        pltpu.sync_copy(ex_hbm.at[pl.ds(base, CB)], ex_v)
        for i in range(CB // L):
            sl = pl.ds(i * L, L)
            r = plsc.load_gather(recip_v, [dst_v[sl]])
            att_v[sl] = ex_v[sl] * r
        pltpu.sync_copy(att_v, att_hbm.at[pl.ds(base, CB)])
        return carry

    lax.fori_loop(0, EPW // CB, chunk, 0)


C4 = 64
NCH = EPW // C4
NCHT = EP // C4
MR = K + 3


@functools.partial(
    pl.kernel,
    out_type=jax.ShapeDtypeStruct((NC, NPA, D), jnp.float32),
    mesh=_MESH,
    compiler_params=pltpu.CompilerParams(needs_layout_passes=False, use_tc_tiling_on_sc=False),
    scratch_types=[
        pltpu.VMEM((2, MR, C4), jnp.int32),
        pltpu.VMEM((2, C4, K * D), jnp.bfloat16),
        pltpu.VMEM((2, C4, D), jnp.float32),
        pltpu.VMEM((2, C4), jnp.int32),
        pltpu.VMEM_SHARED((NPA, D), jnp.float32),
        pltpu.SemaphoreType.DMA,
        pltpu.SemaphoreType.DMA,
        pltpu.SemaphoreType.DMA,
        pltpu.SemaphoreType.DMA,
        pltpu.SemaphoreType.DMA,
        pltpu.SemaphoreType.DMA,
    ],
)
def _k4(meta_hbm, zk_hbm, zrows_hbm,
        opart_hbm,
        meta_v, rows_v, m_v, dstq_v, acc_sc,
        sem_ma, sem_mb, sem_ga, sem_gb, sem_sa, sem_sb):
    cid = lax.axis_index("c")
    sid = lax.axis_index("s")
    wid = sid * NC + cid
    gbase = wid * NCH
    sem_m = [sem_ma, sem_mb]
    sem_g = [sem_ga, sem_gb]
    sem_s = [sem_sa, sem_sb]

    pltpu.sync_copy(zrows_hbm, acc_sc.at[pl.ds(sid * NPSA, NPSA)])
    plsc.subcore_barrier()

    def meta_issue(x, c):
        pltpu.async_copy(meta_hbm.at[gbase + c], meta_v.at[x], sem_m[x])

    def meta_wait(x):
        pltpu.make_async_copy(meta_hbm.at[gbase], meta_v.at[x], sem_m[x]).wait()

    def gather_issue(x):
        pltpu.async_copy(zk_hbm.at[meta_v.at[x, 0]], rows_v.at[x], sem_g[x])

    def gather_wait(x):
        pltpu.make_async_copy(zk_hbm.at[meta_v.at[x, 0]], rows_v.at[x],
                              sem_g[x]).wait()

    def scatter_issue(x):
        pltpu.async_copy(m_v.at[x], acc_sc.at[dstq_v.at[x]], sem_s[x],
                         add=True)

    def scatter_wait(x):
        pltpu.make_async_copy(m_v.at[x], acc_sc.at[dstq_v.at[x]],
                              sem_s[x]).wait()

    def compute(x):
        def group(i, inner):
            sl = pl.ds(i * L, L)
            a1 = plsc.bitcast(meta_v[x, 6, sl], jnp.float32) + 1.0
            c0 = plsc.bitcast(meta_v[x, 2, sl], jnp.float32) * a1
            c1 = plsc.bitcast(meta_v[x, 3, sl], jnp.float32) * a1
            c2 = plsc.bitcast(meta_v[x, 4, sl], jnp.float32) * a1
            c3 = plsc.bitcast(meta_v[x, 5, sl], jnp.float32) * a1
            dstq_v[x, sl] = meta_v[x, 1, sl]
            for t in range(L):
                e = i * L + t
                s0, s1, s2, s3 = c0[t], c1[t], c2[t], c3[t]
                for q in range(D // (2 * L)):
                    r0 = rows_v[x, e, pl.ds(q * 2 * L, 2 * L)]
                    r1 = rows_v[x, e, pl.ds(D + q * 2 * L, 2 * L)]
                    r2 = rows_v[x, e, pl.ds(2 * D + q * 2 * L, 2 * L)]
                    r3 = rows_v[x, e, pl.ds(3 * D + q * 2 * L, 2 * L)]
                    a0lo, a0hi = plsc.unpack(r0, format=plsc.PackFormat.INTERLEAVED)
                    a1lo, a1hi = plsc.unpack(r1, format=plsc.PackFormat.INTERLEAVED)
                    a2lo, a2hi = plsc.unpack(r2, format=plsc.PackFormat.INTERLEAVED)
                    a3lo, a3hi = plsc.unpack(r3, format=plsc.PackFormat.INTERLEAVED)
                    mlo = s0 * a0lo + s1 * a1lo + s2 * a2lo + s3 * a3lo
                    mhi = s0 * a0hi + s1 * a1hi + s2 * a2hi + s3 * a3hi
                    m_v[x, e, pl.ds(q * 2 * L, L)] = mlo
                    m_v[x, e, pl.ds(q * 2 * L + L, L)] = mhi
            return inner

        lax.fori_loop(0, C4 // L, group, 0)

    meta_issue(0, 0)
    meta_issue(1, 1)
    meta_wait(0)
    gather_issue(0)

    def pair(h, carry):
        for p in range(2):
            c = 2 * h + p
            x = p
            xn = 1 - p
            meta_wait(xn)
            gather_issue(xn)
            gather_wait(x)

            @pl.when(h > 0)
            def _():
                scatter_wait(x)

            compute(x)
            scatter_issue(x)
            meta_issue(x, jnp.minimum(c + 2, NCH - 1))
        return carry

    lax.fori_loop(0, NCH // 2, pair, 0)

    gather_wait(0)
    scatter_wait(0)
    scatter_wait(1)
    meta_wait(1)

    plsc.subcore_barrier()
    pltpu.sync_copy(acc_sc.at[pl.ds(sid * NPSA, NPSA)],
                    opart_hbm.at[cid, pl.ds(sid * NPSA, NPSA)])


def _k5_body(p_ref, z4_ref, b_ref, o_ref):
    o_ref[...] = p_ref[0] + p_ref[1] + z4_ref[...] + b_ref[...]


def _k5(opart, z4, bias2d):
    return pl.pallas_call(
        _k5_body,
        grid=(N // ROWS_TC,),
        in_specs=[
            pl.BlockSpec((NC, ROWS_TC, D), lambda i: (0, i, 0)),
            pl.BlockSpec((ROWS_TC, D), lambda i: (i, 0)),
            pl.BlockSpec((1, D), lambda i: (0, 0)),
        ],
        out_specs=pl.BlockSpec((ROWS_TC, D), lambda i: (i, 0)),
        out_shape=jax.ShapeDtypeStruct((N, D), jnp.float32),
    )(opart, z4, bias2d)


def kernel(x, edge_index, edge_attr, weight, bias, attention_vector):
    src = edge_index[0].astype(jnp.int32)
    dst = edge_index[1].astype(jnp.int32)
    pad_e = EP - E
    src_p = jnp.concatenate([src, jnp.zeros((pad_e,), jnp.int32)])
    dst_p = jnp.concatenate([dst, jnp.full((pad_e,), DUMP, jnp.int32)])
    eat = jnp.concatenate(
        [edge_attr.T.astype(jnp.float32), jnp.zeros((K, pad_e), jnp.float32)],
        axis=1)
    perm = jnp.arange(K * D).reshape(K * D // 32, 2, L).transpose(0, 2, 1)
    perm = perm.reshape(K * D)
    wcat = jnp.concatenate(
        [
            weight[:K].transpose(1, 0, 2).reshape(D, K * D)[:, perm],
            weight[K],
            attention_vector.astype(jnp.float32),
            jnp.zeros((D, D - 1), jnp.float32),
        ],
        axis=1)

    zk, aux = _k1(x, wcat)
    z4 = aux[:, :D]
    s_p = jnp.concatenate([aux[:, D], jnp.zeros((NPAD - N,), jnp.float32)])

    ex, dpart = _k2(src_p, dst_p, eat, s_p, jnp.zeros((NPAD,), jnp.float32))
    recip = _k3(dpart)
    att = _k3b(dst_p, ex, recip)
    meta = jnp.concatenate(
        [
            src_p[None, :],
            dst_p[None, :],
            lax.bitcast_convert_type(eat, jnp.int32),
            lax.bitcast_convert_type(att, jnp.int32)[None, :],
        ],
        axis=0).reshape(MR, NCHT, C4).transpose(1, 0, 2)
    opart = _k4(meta, zk, jnp.zeros((NPSA, D), jnp.float32))
    out = _k5(opart, z4, bias.reshape(1, D).astype(jnp.float32))
    return out, att[:E]


# speedup vs baseline: 1.3577x; 1.3577x over previous
import functools

import jax
import jax.numpy as jnp
from jax import lax
from jax.experimental import pallas as pl
from jax.experimental.pallas import tpu as pltpu
from jax.experimental.pallas import tpu_sc as plsc

N = 10000
D = 128
K = 4
E = 320000

NC = 2
NS = 16
NW = NC * NS
L = 16

NPAD = 10240
EP = 327680
EPW = EP // NW
C = 128
NCHUNK = EPW // C
NPS = NPAD // NS
NPW = NPAD // NW

_MESH = plsc.VectorSubcoreMesh(core_axis_name="c", subcore_axis_name="s")

ROWS_TC = 2000


NPA = 10016
NPSA = NPA // NS
DUMP = NPA - 1


def _k1_body(x_ref, w_ref, zk_ref, aux_ref):
    z = jnp.dot(x_ref[...], w_ref[...], preferred_element_type=jnp.float32)
    zk_ref[...] = z[:, : K * D].astype(jnp.bfloat16)
    aux_ref[...] = z[:, K * D :]


def _k1(x, wcat):
    return pl.pallas_call(
        _k1_body,
        grid=(N // ROWS_TC,),
        in_specs=[
            pl.BlockSpec((ROWS_TC, D), lambda i: (i, 0)),
            pl.BlockSpec((D, 6 * D), lambda i: (0, 0)),
        ],
        out_specs=[
            pl.BlockSpec((ROWS_TC, K * D), lambda i: (i, 0)),
            pl.BlockSpec((ROWS_TC, 2 * D), lambda i: (i, 0)),
        ],
        out_shape=[
            jax.ShapeDtypeStruct((N, K * D), jnp.bfloat16),
            jax.ShapeDtypeStruct((N, 2 * D), jnp.float32),
        ],
    )(x, wcat)


C2 = 256
NCH2 = EPW // C2
NCHT2 = EP // C2


@functools.partial(
    pl.kernel,
    out_type=(
        jax.ShapeDtypeStruct((EP,), jnp.float32),
        jax.ShapeDtypeStruct((NW, NPAD), jnp.float32),
    ),
    mesh=_MESH,
    compiler_params=pltpu.CompilerParams(needs_layout_passes=False, use_tc_tiling_on_sc=False),
    scratch_types=[
        pltpu.VMEM((NPAD,), jnp.float32),
        pltpu.VMEM((NPAD,), jnp.float32),
        pltpu.VMEM((2, K + 2, C2), jnp.int32),
        pltpu.VMEM((2, C2), jnp.float32),
        pltpu.SemaphoreType.DMA,
        pltpu.SemaphoreType.DMA,
        pltpu.SemaphoreType.DMA,
        pltpu.SemaphoreType.DMA,
    ],
)
def _k2(meta2_hbm, s_hbm, zed_hbm,
        ex_hbm, dpart_hbm,
        s_v, den_v, meta2_v, ex_v,
        sem_ma, sem_mb, sem_oa, sem_ob):
    cid = lax.axis_index("c")
    sid = lax.axis_index("s")
    wid = sid * NC + cid
    gbase = wid * NCH2
    ebase = wid * EPW
    sem_m = [sem_ma, sem_mb]
    sem_o = [sem_oa, sem_ob]
    pltpu.sync_copy(s_hbm, s_v)
    pltpu.sync_copy(zed_hbm, den_v)

    def meta_issue(x, c):
        pltpu.async_copy(meta2_hbm.at[gbase + c], meta2_v.at[x], sem_m[x])

    def meta_wait(x):
        pltpu.make_async_copy(meta2_hbm.at[gbase], meta2_v.at[x],
                              sem_m[x]).wait()

    def out_issue(x, c):
        pltpu.async_copy(ex_v.at[x], ex_hbm.at[pl.ds(ebase + c * C2, C2)],
                         sem_o[x])

    def out_wait(x):
        pltpu.make_async_copy(ex_v.at[x], ex_hbm.at[pl.ds(ebase, C2)],
                              sem_o[x]).wait()

    meta_issue(0, 0)
    meta_issue(1, 1)

    def pair(h, carry):
        for p in range(2):
            c = 2 * h + p
            x = p
            meta_wait(x)

            @pl.when(h > 0)
            def _():
                out_wait(x)

            def group(i, inner):
                sl = pl.ds(i * L, L)
                sg = plsc.load_gather(s_v, [meta2_v[x, 0, sl]])
                ebar = (plsc.bitcast(meta2_v[x, 2, sl], jnp.float32)
                        + plsc.bitcast(meta2_v[x, 3, sl], jnp.float32)
                        + plsc.bitcast(meta2_v[x, 4, sl], jnp.float32)
                        + plsc.bitcast(meta2_v[x, 5, sl], jnp.float32)) * 0.25
                ex = jnp.exp(ebar * sg)
                ex_v[x, sl] = ex
                plsc.addupdate_scatter(den_v, [meta2_v[x, 1, sl]], ex)
                return inner

            lax.fori_loop(0, C2 // L, group, 0)
            out_issue(x, c)
            meta_issue(x, jnp.minimum(c + 2, NCH2 - 1))
        return carry

    lax.fori_loop(0, NCH2 // 2, pair, 0)
    out_wait(0)
    out_wait(1)
    meta_wait(0)
    meta_wait(1)
    pltpu.sync_copy(den_v, dpart_hbm.at[wid])


@functools.partial(
    pl.kernel,
    out_type=jax.ShapeDtypeStruct((NPAD,), jnp.float32),
    mesh=_MESH,
    compiler_params=pltpu.CompilerParams(needs_layout_passes=False, use_tc_tiling_on_sc=False),
    scratch_types=[
        pltpu.VMEM((NW, NPW), jnp.float32),
        pltpu.VMEM((NPW,), jnp.float32),
    ],
)
def _k3(dpart_hbm, recip_hbm, part_v, acc_v):
    wid = lax.axis_index("s") * NC + lax.axis_index("c")
    for j in range(NW):
        pltpu.sync_copy(dpart_hbm.at[j, pl.ds(wid * NPW, NPW)], part_v.at[j])
    for i in range(NPW // L):
        sl = pl.ds(i * L, L)
        a = part_v[0, sl]
        for j in range(1, NW):
            a = a + part_v[j, sl]
        acc_v[sl] = 1.0 / (a + 1e-16)
    pltpu.sync_copy(acc_v, recip_hbm.at[pl.ds(wid * NPW, NPW)])


CB = 512


@functools.partial(
    pl.kernel,
    out_type=jax.ShapeDtypeStruct((EP,), jnp.float32),
    mesh=_MESH,
    compiler_params=pltpu.CompilerParams(needs_layout_passes=False, use_tc_tiling_on_sc=False),
    scratch_types=[
        pltpu.VMEM((NPAD,), jnp.float32),
        pltpu.VMEM((CB,), jnp.int32),
        pltpu.VMEM((CB,), jnp.float32),
        pltpu.VMEM((CB,), jnp.float32),
    ],
)
def _k3b(dst_hbm, ex_hbm, recip_hbm, att_hbm, recip_v, dst_v, ex_v, att_v):
    wid = lax.axis_index("s") * NC + lax.axis_index("c")
    pltpu.sync_copy(recip_hbm, recip_v)

    def chunk(g, carry):
        base = wid * EPW + g * CB
        pltpu.sync_copy(dst_hbm.at[pl.ds(base, CB)], dst_v)
        pltpu.sync_copy(ex_hbm.at[pl.ds(base, CB)], ex_v)
        for i in range(CB // L):
            sl = pl.ds(i * L, L)
            r = plsc.load_gather(recip_v, [dst_v[sl]])
            att_v[sl] = ex_v[sl] * r
        pltpu.sync_copy(att_v, att_hbm.at[pl.ds(base, CB)])
        return carry

    lax.fori_loop(0, EPW // CB, chunk, 0)


C4 = 64
NCH = EPW // C4
NCHT = EP // C4
MR = K + 3


@functools.partial(
    pl.kernel,
    out_type=jax.ShapeDtypeStruct((NC, NPA, D), jnp.float32),
    mesh=_MESH,
    compiler_params=pltpu.CompilerParams(needs_layout_passes=False, use_tc_tiling_on_sc=False),
    scratch_types=[
        pltpu.VMEM((2, MR, C4), jnp.int32),
        pltpu.VMEM((2, C4, K * D), jnp.bfloat16),
        pltpu.VMEM((2, C4, D), jnp.float32),
        pltpu.VMEM((2, C4), jnp.int32),
        pltpu.VMEM_SHARED((NPA, D), jnp.float32),
        pltpu.SemaphoreType.DMA,
        pltpu.SemaphoreType.DMA,
        pltpu.SemaphoreType.DMA,
        pltpu.SemaphoreType.DMA,
        pltpu.SemaphoreType.DMA,
        pltpu.SemaphoreType.DMA,
    ],
)
def _k4(meta_hbm, zk_hbm, zrows_hbm,
        opart_hbm,
        meta_v, rows_v, m_v, dstq_v, acc_sc,
        sem_ma, sem_mb, sem_ga, sem_gb, sem_sa, sem_sb):
    cid = lax.axis_index("c")
    sid = lax.axis_index("s")
    wid = sid * NC + cid
    gbase = wid * NCH
    sem_m = [sem_ma, sem_mb]
    sem_g = [sem_ga, sem_gb]
    sem_s = [sem_sa, sem_sb]

    pltpu.sync_copy(zrows_hbm, acc_sc.at[pl.ds(sid * NPSA, NPSA)])
    plsc.subcore_barrier()

    def meta_issue(x, c):
        pltpu.async_copy(meta_hbm.at[gbase + c], meta_v.at[x], sem_m[x])

    def meta_wait(x):
        pltpu.make_async_copy(meta_hbm.at[gbase], meta_v.at[x], sem_m[x]).wait()

    def gather_issue(x):
        pltpu.async_copy(zk_hbm.at[meta_v.at[x, 0]], rows_v.at[x], sem_g[x])

    def gather_wait(x):
        pltpu.make_async_copy(zk_hbm.at[meta_v.at[x, 0]], rows_v.at[x],
                              sem_g[x]).wait()

    def scatter_issue(x):
        pltpu.async_copy(m_v.at[x], acc_sc.at[dstq_v.at[x]], sem_s[x],
                         add=True)

    def scatter_wait(x):
        pltpu.make_async_copy(m_v.at[x], acc_sc.at[dstq_v.at[x]],
                              sem_s[x]).wait()

    def compute(x):
        def group(i, inner):
            sl = pl.ds(i * L, L)
            a1 = plsc.bitcast(meta_v[x, 6, sl], jnp.float32) + 1.0
            c0 = plsc.bitcast(meta_v[x, 2, sl], jnp.float32) * a1
            c1 = plsc.bitcast(meta_v[x, 3, sl], jnp.float32) * a1
            c2 = plsc.bitcast(meta_v[x, 4, sl], jnp.float32) * a1
            c3 = plsc.bitcast(meta_v[x, 5, sl], jnp.float32) * a1
            dstq_v[x, sl] = meta_v[x, 1, sl]
            for t in range(L):
                e = i * L + t
                s0, s1, s2, s3 = c0[t], c1[t], c2[t], c3[t]
                for q in range(D // (2 * L)):
                    r0 = rows_v[x, e, pl.ds(q * 2 * L, 2 * L)]
                    r1 = rows_v[x, e, pl.ds(D + q * 2 * L, 2 * L)]
                    r2 = rows_v[x, e, pl.ds(2 * D + q * 2 * L, 2 * L)]
                    r3 = rows_v[x, e, pl.ds(3 * D + q * 2 * L, 2 * L)]
                    a0lo, a0hi = plsc.unpack(r0, format=plsc.PackFormat.INTERLEAVED)
                    a1lo, a1hi = plsc.unpack(r1, format=plsc.PackFormat.INTERLEAVED)
                    a2lo, a2hi = plsc.unpack(r2, format=plsc.PackFormat.INTERLEAVED)
                    a3lo, a3hi = plsc.unpack(r3, format=plsc.PackFormat.INTERLEAVED)
                    mlo = s0 * a0lo + s1 * a1lo + s2 * a2lo + s3 * a3lo
                    mhi = s0 * a0hi + s1 * a1hi + s2 * a2hi + s3 * a3hi
                    m_v[x, e, pl.ds(q * 2 * L, L)] = mlo
                    m_v[x, e, pl.ds(q * 2 * L + L, L)] = mhi
            return inner

        lax.fori_loop(0, C4 // L, group, 0)

    meta_issue(0, 0)
    meta_issue(1, 1)
    meta_wait(0)
    gather_issue(0)

    def pair(h, carry):
        for p in range(2):
            c = 2 * h + p
            x = p
            xn = 1 - p
            meta_wait(xn)
            gather_issue(xn)
            gather_wait(x)

            @pl.when(h > 0)
            def _():
                scatter_wait(x)

            compute(x)
            scatter_issue(x)
            meta_issue(x, jnp.minimum(c + 2, NCH - 1))
        return carry

    lax.fori_loop(0, NCH // 2, pair, 0)

    gather_wait(0)
    scatter_wait(0)
    scatter_wait(1)
    meta_wait(1)

    plsc.subcore_barrier()
    pltpu.sync_copy(acc_sc.at[pl.ds(sid * NPSA, NPSA)],
                    opart_hbm.at[cid, pl.ds(sid * NPSA, NPSA)])


def _k5_body(p_ref, z4_ref, b_ref, o_ref):
    o_ref[...] = p_ref[0] + p_ref[1] + z4_ref[...] + b_ref[...]


def _k5(opart, z4, bias2d):
    return pl.pallas_call(
        _k5_body,
        grid=(N // ROWS_TC,),
        in_specs=[
            pl.BlockSpec((NC, ROWS_TC, D), lambda i: (0, i, 0)),
            pl.BlockSpec((ROWS_TC, D), lambda i: (i, 0)),
            pl.BlockSpec((1, D), lambda i: (0, 0)),
        ],
        out_specs=pl.BlockSpec((ROWS_TC, D), lambda i: (i, 0)),
        out_shape=jax.ShapeDtypeStruct((N, D), jnp.float32),
    )(opart, z4, bias2d)


def kernel(x, edge_index, edge_attr, weight, bias, attention_vector):
    src = edge_index[0].astype(jnp.int32)
    dst = edge_index[1].astype(jnp.int32)
    pad_e = EP - E
    src_p = jnp.concatenate([src, jnp.zeros((pad_e,), jnp.int32)])
    dst_p = jnp.concatenate([dst, jnp.full((pad_e,), DUMP, jnp.int32)])
    eat = jnp.concatenate(
        [edge_attr.T.astype(jnp.float32), jnp.zeros((K, pad_e), jnp.float32)],
        axis=1)
    perm = jnp.arange(K * D).reshape(K * D // 32, 2, L).transpose(0, 2, 1)
    perm = perm.reshape(K * D)
    wcat = jnp.concatenate(
        [
            weight[:K].transpose(1, 0, 2).reshape(D, K * D)[:, perm],
            weight[K],
            attention_vector.astype(jnp.float32),
            jnp.zeros((D, D - 1), jnp.float32),
        ],
        axis=1)

    zk, aux = _k1(x, wcat)
    z4 = aux[:, :D]
    s_p = jnp.concatenate([aux[:, D], jnp.zeros((NPAD - N,), jnp.float32)])

    eat_i = lax.bitcast_convert_type(eat, jnp.int32)
    meta2 = jnp.concatenate(
        [src_p[None, :], dst_p[None, :], eat_i],
        axis=0).reshape(K + 2, NCHT2, C2).transpose(1, 0, 2)
    ex, dpart = _k2(meta2, s_p, jnp.zeros((NPAD,), jnp.float32))
    recip = _k3(dpart)
    att = _k3b(dst_p, ex, recip)
    meta = jnp.concatenate(
        [
            src_p[None, :],
            dst_p[None, :],
            eat_i,
            lax.bitcast_convert_type(att, jnp.int32)[None, :],
        ],
        axis=0).reshape(MR, NCHT, C4).transpose(1, 0, 2)
    opart = _k4(meta, zk, jnp.zeros((NPSA, D), jnp.float32))
    out = _k5(opart, z4, bias.reshape(1, D).astype(jnp.float32))
    return out, att[:E]
